# Initial kernel scaffold; baseline (speedup 1.0000x reference)
import jax
import jax.numpy as jnp
from jax.experimental import pallas as pl


def kernel(feature, edge_index, adj_values, weight_b, W1, b1, W2, b2):
    raise NotImplementedError("write your pallas kernel here")


# speedup vs baseline: 2.4329x; 2.4329x over previous
import functools
import math

import jax
import jax.numpy as jnp
from jax import lax
from jax.experimental import pallas as pl
from jax.experimental.pallas import tpu as pltpu
from jax.experimental.pallas import tpu_sc as plsc

NC = 2
NS = 16
CH = 128
G = 8


def _prep_call(adjT, wb, feature, W1):
    Rr, E = adjT.shape
    N, F = feature.shape
    OUT = W1.shape[1]

    def body(adjT_ref, wb_ref, x_ref, w1_ref, ewT_ref, s1_ref):
        ewT_ref[...] = jnp.sum(adjT_ref[...] * wb_ref[...], axis=0,
                               keepdims=True)
        s1_ref[...] = jnp.dot(x_ref[...], w1_ref[...],
                              preferred_element_type=jnp.float32)

    return pl.pallas_call(
        body,
        out_shape=[
            jax.ShapeDtypeStruct((1, E), jnp.float32),
            jax.ShapeDtypeStruct((N, OUT), jnp.float32),
        ],
    )(adjT, wb, feature, W1)


def _mid_call(parts, b1, W2):
    _, N, OUT = parts.shape

    def body(p_ref, b1_ref, w2_ref, u1_ref, s2_ref):
        u1 = p_ref[0] + p_ref[1] + b1_ref[...]
        u1_ref[...] = u1
        s2_ref[...] = jnp.dot(u1, w2_ref[...],
                              preferred_element_type=jnp.float32)

    return pl.pallas_call(
        body,
        out_shape=[
            jax.ShapeDtypeStruct((N, OUT), jnp.float32),
            jax.ShapeDtypeStruct((N, OUT), jnp.float32),
        ],
    )(parts, b1, W2)


def _final_call(parts, U1, b2):
    _, N, OUT = parts.shape

    def body(p_ref, u1_ref, b2_ref, out_ref):
        u2 = p_ref[0] + p_ref[1] + b2_ref[...]
        out_ref[...] = (u1_ref[...] + u2) * 0.5

    return pl.pallas_call(
        body,
        out_shape=jax.ShapeDtypeStruct((N, OUT), jnp.float32),
    )(parts, U1, b2)


@functools.lru_cache(maxsize=None)
def _make_spmm(NP, OUT, ng):
    rows_per_tile = NP // NS
    nq = OUT // 16
    mesh = plsc.VectorSubcoreMesh(core_axis_name="c", subcore_axis_name="s",
                                  num_cores=NC, num_subcores=NS)

    @functools.partial(
        pl.kernel,
        out_type=jax.ShapeDtypeStruct((NC, NP, OUT), jnp.float32),
        mesh=mesh,
        scratch_types=[
            pltpu.VMEM((G, CH), jnp.int32),
            pltpu.VMEM((G, CH), jnp.int32),
            pltpu.VMEM((G, CH), jnp.float32),
            pltpu.VMEM((CH, OUT), jnp.float32),
            pltpu.VMEM_SHARED((NP, OUT), jnp.float32),
            pltpu.SemaphoreType.DMA,
        ],
        compiler_params=pltpu.CompilerParams(use_tc_tiling_on_sc=False),
    )
    def spmm(src_hbm, dst_hbm, w_hbm, x_hbm, z_hbm, out_hbm,
             src_v, dst_v, w_v, rows_v, acc_sh, sem):
        c = lax.axis_index("c")
        s = lax.axis_index("s")
        row0 = s * rows_per_tile
        pltpu.sync_copy(z_hbm, acc_sh.at[pl.ds(row0, rows_per_tile)])
        plsc.subcore_barrier()

        def group(g, carry):
            pltpu.sync_copy(src_hbm.at[c, s, pl.ds(g * G, G)], src_v)
            pltpu.sync_copy(dst_hbm.at[c, s, pl.ds(g * G, G)], dst_v)
            pltpu.sync_copy(w_hbm.at[c, s, pl.ds(g * G, G)], w_v)
            for j in range(G):
                pltpu.async_copy(x_hbm.at[src_v.at[j]], rows_v, sem).wait()

                def edge16(eg, carry2):
                    wv16 = w_v[j, pl.ds(eg * 16, 16)]
                    for i in range(16):
                        wspl = jnp.full((16,), wv16[i], jnp.float32)
                        e = eg * 16 + i
                        for q in range(nq):
                            sl = pl.ds(q * 16, 16)
                            rows_v[e, sl] = rows_v[e, sl] * wspl
                    return carry2

                lax.fori_loop(0, CH // 16, edge16, 0)
                pltpu.sync_copy(rows_v, acc_sh.at[dst_v.at[j]], add=True)
            return carry

        lax.fori_loop(0, ng, group, 0)
        plsc.subcore_barrier()
        pltpu.sync_copy(acc_sh.at[pl.ds(row0, rows_per_tile)],
                        out_hbm.at[c, pl.ds(row0, rows_per_tile)])

    return spmm


def kernel(feature, edge_index, adj_values, weight_b, W1, b1, W2, b2):
    N, F = feature.shape
    E = edge_index.shape[1]
    OUT = W1.shape[1]

    ei = edge_index.astype(jnp.int32)
    adjT = adj_values.T

    ewT, S1 = _prep_call(adjT, weight_b, feature, W1)
    ew = ewT[0]

    src = jnp.concatenate([ei[1], ei[0]])
    dst = jnp.concatenate([ei[0], ei[1]])
    w = jnp.concatenate([ew, ew])

    per_tile = math.ceil(2 * E / (NC * NS * CH * G)) * CH * G
    ng = per_tile // (CH * G)
    ep = NC * NS * per_tile
    pad = ep - 2 * E
    src = jnp.pad(src, (0, pad)).reshape(NC, NS, per_tile // CH, CH)
    dst = jnp.pad(dst, (0, pad)).reshape(NC, NS, per_tile // CH, CH)
    w = jnp.pad(w, (0, pad)).reshape(NC, NS, per_tile // CH, CH)

    NP = math.ceil(N / (8 * NS)) * 8 * NS
    zeros = jnp.zeros((NP // NS, OUT), jnp.float32)
    spmm = _make_spmm(NP, OUT, ng)

    p1 = spmm(src, dst, w, S1, zeros)
    U1, S2 = _mid_call(p1, b1, W2)
    p2 = spmm(src, dst, w, S2, zeros)
    return _final_call(p2, U1, b2)[:N]


# speedup vs baseline: 2.8512x; 1.1719x over previous
import functools
import math

import jax
import jax.numpy as jnp
from jax import lax
from jax.experimental import pallas as pl
from jax.experimental.pallas import tpu as pltpu
from jax.experimental.pallas import tpu_sc as plsc

NC = 2
NS = 16
CH = 128
G = 8


def _prep_call(adjT, wb, feature, W1):
    Rr, E = adjT.shape
    N, F = feature.shape
    OUT = W1.shape[1]

    def body(adjT_ref, wb_ref, x_ref, w1_ref, ewT_ref, s1_ref):
        ewT_ref[...] = jnp.sum(adjT_ref[...] * wb_ref[...], axis=0,
                               keepdims=True)
        s1_ref[...] = jnp.dot(x_ref[...], w1_ref[...],
                              preferred_element_type=jnp.float32)

    return pl.pallas_call(
        body,
        out_shape=[
            jax.ShapeDtypeStruct((1, E), jnp.float32),
            jax.ShapeDtypeStruct((N, OUT), jnp.float32),
        ],
    )(adjT, wb, feature, W1)


def _mid_call(parts, b1, W2):
    _, N, OUT = parts.shape

    def body(p_ref, b1_ref, w2_ref, u1_ref, s2_ref):
        u1 = p_ref[0] + p_ref[1] + b1_ref[...]
        u1_ref[...] = u1
        s2_ref[...] = jnp.dot(u1, w2_ref[...],
                              preferred_element_type=jnp.float32)

    return pl.pallas_call(
        body,
        out_shape=[
            jax.ShapeDtypeStruct((N, OUT), jnp.float32),
            jax.ShapeDtypeStruct((N, OUT), jnp.float32),
        ],
    )(parts, b1, W2)


def _final_call(parts, U1, b2):
    _, N, OUT = parts.shape

    def body(p_ref, u1_ref, b2_ref, out_ref):
        u2 = p_ref[0] + p_ref[1] + b2_ref[...]
        out_ref[...] = (u1_ref[...] + u2) * 0.5

    return pl.pallas_call(
        body,
        out_shape=jax.ShapeDtypeStruct((N, OUT), jnp.float32),
    )(parts, U1, b2)


NBUF = 4


@functools.lru_cache(maxsize=None)
def _make_spmm(NP, OUT, nch):
    rows_per_tile = NP // NS
    nq = OUT // 16
    mesh = plsc.VectorSubcoreMesh(core_axis_name="c", subcore_axis_name="s",
                                  num_cores=NC, num_subcores=NS)
    NH = 2
    assert nch % (NH * NBUF) == 0
    hch = nch // NH

    @functools.partial(
        pl.kernel,
        out_type=jax.ShapeDtypeStruct((NC, NP, OUT), jnp.float32),
        mesh=mesh,
        scratch_types=[
            pltpu.VMEM((hch, CH), jnp.int32),
            pltpu.VMEM((hch, CH), jnp.int32),
            pltpu.VMEM((hch, CH), jnp.float32),
            [pltpu.VMEM((CH, OUT), jnp.float32) for _ in range(NBUF)],
            pltpu.VMEM_SHARED((NP, OUT), jnp.float32),
            [pltpu.SemaphoreType.DMA for _ in range(NBUF)],
            [pltpu.SemaphoreType.DMA for _ in range(NBUF)],
            pltpu.SemaphoreType.DMA,
        ],
        compiler_params=pltpu.CompilerParams(use_tc_tiling_on_sc=False),
    )
    def spmm(src_hbm, dst_hbm, w_hbm, x_hbm, z_hbm, out_hbm,
             src_v, dst_v, w_v, rows, acc_sh, sg, ss, sem):
        c = lax.axis_index("c")
        s = lax.axis_index("s")
        row0 = s * rows_per_tile
        pltpu.sync_copy(z_hbm, acc_sh.at[pl.ds(row0, rows_per_tile)])
        plsc.subcore_barrier()

        zv = jnp.zeros((16,), jnp.float32)

        def scale(buf, ch):
            def edge16(eg, carry2):
                wv16 = w_v[ch, pl.ds(eg * 16, 16)]
                for i in range(16):
                    wspl = jnp.full((16,), wv16[i], jnp.float32)
                    e = eg * 16 + i
                    for q in range(nq):
                        sl = pl.ds(q * 16, 16)
                        buf[e, sl] = buf[e, sl] * wspl
                return carry2

            lax.fori_loop(0, CH // 16, edge16, 0)

        def drain(sem, b):
            pltpu.make_async_copy(x_hbm.at[pl.ds(0, CH)], rows[b], sem).wait()

        for h in range(NH):
            pltpu.async_copy(src_hbm.at[c, s, pl.ds(h * hch, hch)],
                             src_v, sem).wait()
            pltpu.async_copy(dst_hbm.at[c, s, pl.ds(h * hch, hch)],
                             dst_v, sem).wait()
            pltpu.async_copy(w_hbm.at[c, s, pl.ds(h * hch, hch)],
                             w_v, sem).wait()
            pltpu.async_copy(x_hbm.at[src_v.at[0]], rows[0], sg[0])
            pltpu.async_copy(x_hbm.at[src_v.at[1]], rows[1], sg[1])

            def zrow(e, carry2):
                for b in range(2, NBUF):
                    for q in range(nq):
                        rows[b][e, pl.ds(q * 16, 16)] = zv
                return carry2

            lax.fori_loop(0, CH, zrow, 0)
            for b in range(2, NBUF):
                pltpu.async_copy(rows[b], acc_sh.at[dst_v.at[0]], ss[b],
                                 add=True)

            def step(t, carry):
                for i in range(NBUF):
                    ch = t * NBUF + i
                    b = i
                    bn = (i + 2) % NBUF
                    drain(sg[b], b)
                    scale(rows[b], ch)
                    pltpu.async_copy(rows[b], acc_sh.at[dst_v.at[ch]],
                                     ss[b], add=True)
                    chn = ch + 2 - jnp.where(ch + 2 >= hch, hch, 0)
                    drain(ss[bn], bn)
                    pltpu.async_copy(x_hbm.at[src_v.at[chn]], rows[bn],
                                     sg[bn])
                return carry

            lax.fori_loop(0, hch // NBUF, step, 0)
            drain(sg[0], 0)
            drain(sg[1], 1)
            for b in range(2, NBUF):
                drain(ss[b], b)

        plsc.subcore_barrier()
        pltpu.sync_copy(acc_sh.at[pl.ds(row0, rows_per_tile)],
                        out_hbm.at[c, pl.ds(row0, rows_per_tile)])

    return spmm


def kernel(feature, edge_index, adj_values, weight_b, W1, b1, W2, b2):
    N, F = feature.shape
    E = edge_index.shape[1]
    OUT = W1.shape[1]

    ei = edge_index.astype(jnp.int32)
    adjT = adj_values.T

    ewT, S1 = _prep_call(adjT, weight_b, feature, W1)
    ew = ewT[0]

    src = jnp.concatenate([ei[1], ei[0]])
    dst = jnp.concatenate([ei[0], ei[1]])
    w = jnp.concatenate([ew, ew])

    per_tile = math.ceil(2 * E / (NC * NS * CH * 2 * NBUF)) * CH * 2 * NBUF
    nch = per_tile // CH
    ep = NC * NS * per_tile
    pad = ep - 2 * E
    src = jnp.pad(src, (0, pad)).reshape(NC, NS, per_tile // CH, CH)
    dst = jnp.pad(dst, (0, pad)).reshape(NC, NS, per_tile // CH, CH)
    w = jnp.pad(w, (0, pad)).reshape(NC, NS, per_tile // CH, CH)

    NP = math.ceil(N / (8 * NS)) * 8 * NS
    zeros = jnp.zeros((NP // NS, OUT), jnp.float32)
    spmm = _make_spmm(NP, OUT, nch)

    p1 = spmm(src, dst, w, S1, zeros)
    U1, S2 = _mid_call(p1, b1, W2)
    p2 = spmm(src, dst, w, S2, zeros)
    return _final_call(p2, U1, b2)[:N]


# speedup vs baseline: 3.4642x; 1.2150x over previous
import functools
import math

import jax
import jax.numpy as jnp
from jax import lax
from jax.experimental import pallas as pl
from jax.experimental.pallas import tpu as pltpu
from jax.experimental.pallas import tpu_sc as plsc

NC = 2
NS = 16
CH = 128
G = 8


def _prep_call(adjT, wb, feature, W1):
    Rr, E = adjT.shape
    N, F = feature.shape
    OUT = W1.shape[1]

    def body(adjT_ref, wb_ref, x_ref, w1_ref, ewT_ref, s1_ref):
        ewT_ref[...] = jnp.sum(adjT_ref[...] * wb_ref[...], axis=0,
                               keepdims=True)
        s1_ref[...] = jnp.dot(x_ref[...], w1_ref[...],
                              preferred_element_type=jnp.float32)

    return pl.pallas_call(
        body,
        out_shape=[
            jax.ShapeDtypeStruct((1, E), jnp.float32),
            jax.ShapeDtypeStruct((N, OUT), jnp.float32),
        ],
    )(adjT, wb, feature, W1)


def _mid_call(parts, b1, W2):
    _, N, OUT = parts.shape

    def body(p_ref, b1_ref, w2_ref, u1_ref, s2_ref):
        u1 = p_ref[0] + p_ref[1] + b1_ref[...]
        u1_ref[...] = u1
        s2_ref[...] = jnp.dot(u1, w2_ref[...],
                              preferred_element_type=jnp.float32)

    return pl.pallas_call(
        body,
        out_shape=[
            jax.ShapeDtypeStruct((N, OUT), jnp.float32),
            jax.ShapeDtypeStruct((N, OUT), jnp.float32),
        ],
    )(parts, b1, W2)


def _final_call(parts, U1, b2):
    _, N, OUT = parts.shape

    def body(p_ref, u1_ref, b2_ref, out_ref):
        u2 = p_ref[0] + p_ref[1] + b2_ref[...]
        out_ref[...] = (u1_ref[...] + u2) * 0.5

    return pl.pallas_call(
        body,
        out_shape=jax.ShapeDtypeStruct((N, OUT), jnp.float32),
    )(parts, U1, b2)


NBUF = 4


@functools.lru_cache(maxsize=None)
def _make_spmm(NP, OUT, nch):
    rows_per_tile = NP // NS
    nq = OUT // 16
    mesh = plsc.VectorSubcoreMesh(core_axis_name="c", subcore_axis_name="s",
                                  num_cores=NC, num_subcores=NS)
    NH = 2
    assert nch % (NH * NBUF) == 0
    hch = nch // NH

    @functools.partial(
        pl.kernel,
        out_type=jax.ShapeDtypeStruct((NC, NP, OUT), jnp.float32),
        mesh=mesh,
        scratch_types=[
            pltpu.VMEM((hch, CH), jnp.int32),
            pltpu.VMEM((hch, CH), jnp.int32),
            pltpu.VMEM((hch, CH), jnp.float32),
            [pltpu.VMEM((CH, OUT), jnp.float32) for _ in range(NBUF)],
            pltpu.VMEM_SHARED((NP, OUT), jnp.float32),
            [pltpu.SemaphoreType.DMA for _ in range(NBUF)],
            [pltpu.SemaphoreType.DMA for _ in range(NBUF)],
            pltpu.SemaphoreType.DMA,
        ],
        compiler_params=pltpu.CompilerParams(use_tc_tiling_on_sc=False),
    )
    def spmm(src_hbm, dst_hbm, w_hbm, x_hbm, z_hbm, out_hbm,
             src_v, dst_v, w_v, rows, acc_sh, sg, ss, sem):
        c = lax.axis_index("c")
        s = lax.axis_index("s")
        row0 = s * rows_per_tile
        pltpu.sync_copy(z_hbm, acc_sh.at[pl.ds(row0, rows_per_tile)])
        plsc.subcore_barrier()

        zv = jnp.zeros((16,), jnp.float32)

        def scale(buf, ch):
            @plsc.parallel_loop(0, CH // 16, 1, unroll=4)
            def edge16(eg):
                wv16 = w_v[ch, pl.ds(eg * 16, 16)]
                for i in range(16):
                    wspl = jnp.full((16,), wv16[i], jnp.float32)
                    e = eg * 16 + i
                    for q in range(nq):
                        sl = pl.ds(q * 16, 16)
                        buf[e, sl] = buf[e, sl] * wspl

        def drain(sem, b):
            pltpu.make_async_copy(x_hbm.at[pl.ds(0, CH)], rows[b], sem).wait()

        for h in range(NH):
            pltpu.async_copy(src_hbm.at[c, s, pl.ds(h * hch, hch)],
                             src_v, sem).wait()
            pltpu.async_copy(dst_hbm.at[c, s, pl.ds(h * hch, hch)],
                             dst_v, sem).wait()
            pltpu.async_copy(w_hbm.at[c, s, pl.ds(h * hch, hch)],
                             w_v, sem).wait()
            pltpu.async_copy(x_hbm.at[src_v.at[0]], rows[0], sg[0])
            pltpu.async_copy(x_hbm.at[src_v.at[1]], rows[1], sg[1])

            def zrow(e, carry2):
                for b in range(2, NBUF):
                    for q in range(nq):
                        rows[b][e, pl.ds(q * 16, 16)] = zv
                return carry2

            lax.fori_loop(0, CH, zrow, 0)
            for b in range(2, NBUF):
                pltpu.async_copy(rows[b], acc_sh.at[dst_v.at[0]], ss[b],
                                 add=True)

            def step(t, carry):
                for i in range(NBUF):
                    ch = t * NBUF + i
                    b = i
                    bn = (i + 2) % NBUF
                    drain(sg[b], b)
                    scale(rows[b], ch)
                    pltpu.async_copy(rows[b], acc_sh.at[dst_v.at[ch]],
                                     ss[b], add=True)
                    chn = ch + 2 - jnp.where(ch + 2 >= hch, hch, 0)
                    drain(ss[bn], bn)
                    pltpu.async_copy(x_hbm.at[src_v.at[chn]], rows[bn],
                                     sg[bn])
                return carry

            lax.fori_loop(0, hch // NBUF, step, 0)
            drain(sg[0], 0)
            drain(sg[1], 1)
            for b in range(2, NBUF):
                drain(ss[b], b)

        plsc.subcore_barrier()
        pltpu.sync_copy(acc_sh.at[pl.ds(row0, rows_per_tile)],
                        out_hbm.at[c, pl.ds(row0, rows_per_tile)])

    return spmm


def kernel(feature, edge_index, adj_values, weight_b, W1, b1, W2, b2):
    N, F = feature.shape
    E = edge_index.shape[1]
    OUT = W1.shape[1]

    ei = edge_index.astype(jnp.int32)
    adjT = adj_values.T

    ewT, S1 = _prep_call(adjT, weight_b, feature, W1)
    ew = ewT[0]

    src = jnp.concatenate([ei[1], ei[0]])
    dst = jnp.concatenate([ei[0], ei[1]])
    w = jnp.concatenate([ew, ew])

    per_tile = math.ceil(2 * E / (NC * NS * CH * 2 * NBUF)) * CH * 2 * NBUF
    nch = per_tile // CH
    ep = NC * NS * per_tile
    pad = ep - 2 * E
    src = jnp.pad(src, (0, pad)).reshape(NC, NS, per_tile // CH, CH)
    dst = jnp.pad(dst, (0, pad)).reshape(NC, NS, per_tile // CH, CH)
    w = jnp.pad(w, (0, pad)).reshape(NC, NS, per_tile // CH, CH)

    NP = math.ceil(N / (8 * NS)) * 8 * NS
    zeros = jnp.zeros((NP // NS, OUT), jnp.float32)
    spmm = _make_spmm(NP, OUT, nch)

    p1 = spmm(src, dst, w, S1, zeros)
    U1, S2 = _mid_call(p1, b1, W2)
    p2 = spmm(src, dst, w, S2, zeros)
    return _final_call(p2, U1, b2)[:N]
